# baseline (speedup 1.0000x reference)
import jax
import jax.numpy as jnp
from jax import lax
from jax.experimental import pallas as pl
from jax.experimental.pallas import tpu as pltpu
from jax.experimental.pallas import tpu_sc as plsc

_NB = 8
_N = 32768
_L = 16
_T = 4
_Q = _N // _T
_QCHUNK = _Q // _L
_NCHUNK = _N // _L
_RADIX = 1024
_GB = 1024
_HMAX = float((1 << 10) - 1)


def _spread_bits(n):
    n = n & 0x000003FF
    n = (n | (n << 16)) & 0x030000FF
    n = (n | (n << 8)) & 0x0300F00F
    n = (n | (n << 4)) & 0x030C30C3
    n = (n | (n << 2)) & 0x09249249
    return n


def _f32(v):
    return lax.bitcast_convert_type(v, jnp.float32)


def _sc_body(xf, ot, zsh, psha, hsh, mmsh,
             zbuf, qint, qcode, perm, hist, offs, hist4, mmst,
             svals, sidx, sem):
    w = lax.axis_index("c") * 16 + lax.axis_index("s")
    b = w >> 2
    lb = b & 3
    q = w & 3
    iota = lax.iota(jnp.int32, _L)
    zeros = jnp.zeros((_L,), jnp.int32)
    ones = jnp.ones((_L,), jnp.int32)
    qbase = q * _Q

    pltpu.sync_copy(xf.at[b, pl.ds(qbase * 3, _Q * 3)], qint)
    i3 = iota * 3

    def minmax(j, mm):
        mnx, mxx, mny, mxy, mnz, mxz = mm
        s3 = j * (_L * 3)
        fx = _f32(plsc.load_gather(qint, [i3 + s3]))
        fy = _f32(plsc.load_gather(qint, [i3 + (s3 + 1)]))
        fz = _f32(plsc.load_gather(qint, [i3 + (s3 + 2)]))
        return (jnp.minimum(mnx, fx), jnp.maximum(mxx, fx),
                jnp.minimum(mny, fy), jnp.maximum(mxy, fy),
                jnp.minimum(mnz, fz), jnp.maximum(mxz, fz))

    big = jnp.full((_L,), jnp.inf, jnp.float32)
    mm = lax.fori_loop(0, _QCHUNK, minmax, (big, -big, big, -big, big, -big))
    for i in range(6):
        mmst[pl.ds(i * _L, _L)] = lax.bitcast_convert_type(mm[i], jnp.int32)
    pltpu.sync_copy(mmst, mmsh.at[pl.ds((lb * _T + q) * 96, 96)])
    plsc.subcore_barrier()
    pltpu.sync_copy(mmsh.at[pl.ds(lb * _T * 96, _T * 96)],
                    hist4.at[pl.ds(0, _T * 96)])
    gl = []
    for i in range(6):
        red = jnp.minimum if (i % 2 == 0) else jnp.maximum
        acc = _f32(hist4[pl.ds(i * _L, _L)])
        for t in range(1, _T):
            acc = red(acc, _f32(hist4[pl.ds(t * 96 + i * _L, _L)]))
        gl.append(jnp.min(acc) if (i % 2 == 0) else jnp.max(acc))
    bmin_x, bmax_x, bmin_y, bmax_y, bmin_z, bmax_z = gl

    mnxv = jnp.full((_L,), bmin_x, jnp.float32)
    rxv = jnp.full((_L,), bmax_x - bmin_x, jnp.float32)
    mnyv = jnp.full((_L,), bmin_y, jnp.float32)
    ryv = jnp.full((_L,), bmax_y - bmin_y, jnp.float32)
    mnzv = jnp.full((_L,), bmin_z, jnp.float32)
    rzv = jnp.full((_L,), bmax_z - bmin_z, jnp.float32)

    def zero_hist(i, _):
        hist[pl.ds(i * _L, _L)] = zeros
        return 0

    lax.fori_loop(0, _RADIX // _L, zero_hist, 0)

    def quant(j, _):
        s3 = j * (_L * 3)
        fx = _f32(plsc.load_gather(qint, [i3 + s3]))
        fy = _f32(plsc.load_gather(qint, [i3 + (s3 + 1)]))
        fz = _f32(plsc.load_gather(qint, [i3 + (s3 + 2)]))
        px = (_HMAX * (fx - mnxv) / rxv).astype(jnp.int32)
        py = (_HMAX * (fy - mnyv) / ryv).astype(jnp.int32)
        pz = (_HMAX * (fz - mnzv) / rzv).astype(jnp.int32)
        code = (_spread_bits(px) | (_spread_bits(py) << 1)
                | (_spread_bits(pz) << 2))
        qcode[pl.ds(j * _L, _L)] = code
        plsc.addupdate_scatter(hist, [code & (_RADIX - 1)], ones)
        return 0

    lax.fori_loop(0, _QCHUNK, quant, 0)

    pltpu.sync_copy(qcode, zsh.at[pl.ds(lb * _N + qbase, _Q)])
    plsc.subcore_barrier()
    pltpu.sync_copy(zsh.at[pl.ds(lb * _N, _N)], zbuf)

    qsel = jnp.full((_L,), q, jnp.int32)

    def prefix_from_hist4():
        def prefix(t, carry):
            d0 = t * _L
            h0 = hist4[pl.ds(d0, _L)]
            h1 = hist4[pl.ds(_RADIX + d0, _L)]
            h2 = hist4[pl.ds(2 * _RADIX + d0, _L)]
            h3 = hist4[pl.ds(3 * _RADIX + d0, _L)]
            tot = h0 + h1 + h2 + h3
            part = (jnp.where(qsel > 0, h0, zeros)
                    + jnp.where(qsel > 1, h1, zeros)
                    + jnp.where(qsel > 2, h2, zeros))
            incl = plsc.cumsum(tot)
            offs[pl.ds(d0, _L)] = (incl - tot + part
                                   + jnp.full((_L,), carry, jnp.int32))
            hist[pl.ds(d0, _L)] = zeros
            return carry + jnp.max(incl)

        lax.fori_loop(0, _RADIX // _L, prefix, jnp.int32(0))

    def exchange_hist():
        pltpu.sync_copy(hist, hsh.at[pl.ds((lb * _T + q) * _RADIX, _RADIX)])
        plsc.subcore_barrier()
        pltpu.sync_copy(hsh.at[pl.ds(lb * _T * _RADIX, _T * _RADIX)], hist4)

    def hist_sweep(shift):
        def histo(j, _):
            s = j * _L
            old = perm[pl.ds(s, _L)]
            keys = plsc.load_gather(zbuf, [old])
            qcode[pl.ds(s, _L)] = keys
            d = (keys >> shift) & (_RADIX - 1)
            plsc.addupdate_scatter(hist, [d], ones)
            return 0

        lax.fori_loop(0, _QCHUNK, histo, 0)

    def perm_sweep(shift, first, dst_sh):
        nbat = _Q // _GB
        nrow = _GB // 128

        def batch(m, _):
            def group(g, _):
                j = m * (_GB // _L) + g
                s = j * _L
                if first:
                    old = qbase + s + iota
                    keys = qcode[pl.ds(s, _L)]
                else:
                    old = perm[pl.ds(s, _L)]
                    keys = qcode[pl.ds(s, _L)]
                d = (keys >> shift) & (_RADIX - 1)
                base = plsc.load_gather(offs, [d])
                cnt, last = plsc.scan_count(d)
                dest = base + cnt - 1
                row = g >> 3
                col = (g & 7) * _L
                svals[row, pl.ds(col, _L)] = old
                sidx[row, pl.ds(col, _L)] = dest + lb * _N
                plsc.store_scatter(offs, [d], dest + 1, mask=last)
                return 0

            lax.fori_loop(0, _GB // _L, group, 0)
            copies = [pltpu.async_copy(svals.at[r], dst_sh.at[sidx.at[r]],
                                       sem) for r in range(nrow)]
            for cp in copies:
                cp.wait()
            return 0

        lax.fori_loop(0, nbat, batch, 0)
        plsc.subcore_barrier()

    exchange_hist()
    prefix_from_hist4()
    perm_sweep(0, True, psha)
    pltpu.sync_copy(psha.at[pl.ds(lb * _N + qbase, _Q)], perm)
    hist_sweep(10)
    exchange_hist()
    prefix_from_hist4()
    perm_sweep(10, False, zsh)
    pltpu.sync_copy(zsh.at[pl.ds(lb * _N + qbase, _Q)], perm)
    hist_sweep(20)
    exchange_hist()
    prefix_from_hist4()
    perm_sweep(20, False, psha)
    pltpu.sync_copy(psha.at[pl.ds(lb * _N + qbase, _Q)], perm)

    def widx(j, _):
        s3 = j * (_L * 3)
        p3 = perm[pl.ds(j * _L, _L)] * 3
        plsc.store_scatter(zbuf, [i3 + s3], p3)
        plsc.store_scatter(zbuf, [i3 + (s3 + 1)], p3 + 1)
        plsc.store_scatter(zbuf, [i3 + (s3 + 2)], p3 + 2)
        return 0

    lax.fori_loop(0, _QCHUNK, widx, 0)
    nch = (_Q * 3) // 128
    for r0 in range(0, nch, 16):
        copies = [pltpu.async_copy(
            xf.at[b].at[zbuf.at[pl.ds((r0 + i) * 128, 128)]],
            qint.at[pl.ds((r0 + i) * 128, 128)], sem) for i in range(16)]
        for cp in copies:
            cp.wait()
    pltpu.sync_copy(qint, ot.at[b, pl.ds(qbase * 3, _Q * 3)])


@jax.jit
def kernel(x):
    xi = lax.bitcast_convert_type(x, jnp.int32)
    xf = jnp.reshape(xi, (_NB, _N * 3))
    mesh = plsc.VectorSubcoreMesh(core_axis_name="c", subcore_axis_name="s")
    run = pl.kernel(
        _sc_body,
        out_type=jax.ShapeDtypeStruct((_NB, _N * 3), jnp.int32),
        mesh=mesh,
        compiler_params=pltpu.CompilerParams(needs_layout_passes=False,
                                             use_tc_tiling_on_sc=False),
        scratch_types=[
            pltpu.VMEM_SHARED((4 * _N,), jnp.int32),
            pltpu.VMEM_SHARED((4 * _N,), jnp.int32),
            pltpu.VMEM_SHARED((4 * _T * _RADIX,), jnp.int32),
            pltpu.VMEM_SHARED((4 * _T * 96,), jnp.int32),
            pltpu.VMEM((_N,), jnp.int32),
            pltpu.VMEM((_Q * 3,), jnp.int32),
            pltpu.VMEM((_Q,), jnp.int32),
            pltpu.VMEM((_Q,), jnp.int32),
            pltpu.VMEM((_RADIX,), jnp.int32),
            pltpu.VMEM((_RADIX,), jnp.int32),
            pltpu.VMEM((_T * _RADIX,), jnp.int32),
            pltpu.VMEM((96,), jnp.int32),
            pltpu.VMEM((_GB // 128, 128), jnp.int32),
            pltpu.VMEM((_GB // 128, 128), jnp.int32),
            pltpu.SemaphoreType.DMA,
        ],
    )
    outi = run(xf)
    return lax.bitcast_convert_type(jnp.reshape(outi, (_NB, _N, 3)),
                                    jnp.float32)


# speedup vs baseline: 1.5644x; 1.5644x over previous
import jax
import jax.numpy as jnp
from jax import lax
from jax.experimental import pallas as pl
from jax.experimental.pallas import tpu as pltpu
from jax.experimental.pallas import tpu_sc as plsc

_NB = 8
_N = 32768
_L = 16
_T = 4
_Q = _N // _T
_QCHUNK = _Q // _L
_NCHUNK = _N // _L
_RADIX = 1024
_GB = 1024
_HMAX = float((1 << 10) - 1)


def _spread_bits(n):
    n = n & 0x000003FF
    n = (n | (n << 16)) & 0x030000FF
    n = (n | (n << 8)) & 0x0300F00F
    n = (n | (n << 4)) & 0x030C30C3
    n = (n | (n << 2)) & 0x09249249
    return n


def _f32(v):
    return lax.bitcast_convert_type(v, jnp.float32)


def _sc_body(xt, ot, zsh, psha, pshb, hsh, mmsh,
             zbuf, qx, qy, qz, qint, perm, hist, offs, hist4, mmst,
             svals, sidx, sem):
    w = lax.axis_index("c") * 16 + lax.axis_index("s")
    b = w >> 2
    lb = b & 3
    q = w & 3
    iota = lax.iota(jnp.int32, _L)
    zeros = jnp.zeros((_L,), jnp.int32)
    ones = jnp.ones((_L,), jnp.int32)
    qbase = q * _Q

    pltpu.sync_copy(xt.at[b, 0, pl.ds(qbase, _Q)], qx)
    pltpu.sync_copy(xt.at[b, 1, pl.ds(qbase, _Q)], qy)
    pltpu.sync_copy(xt.at[b, 2, pl.ds(qbase, _Q)], qz)

    def minmax(j, mm):
        mnx, mxx, mny, mxy, mnz, mxz = mm
        s = j * _L
        fx = _f32(qx[pl.ds(s, _L)])
        fy = _f32(qy[pl.ds(s, _L)])
        fz = _f32(qz[pl.ds(s, _L)])
        return (jnp.minimum(mnx, fx), jnp.maximum(mxx, fx),
                jnp.minimum(mny, fy), jnp.maximum(mxy, fy),
                jnp.minimum(mnz, fz), jnp.maximum(mxz, fz))

    big = jnp.full((_L,), jnp.inf, jnp.float32)
    mm = lax.fori_loop(0, _QCHUNK, minmax, (big, -big, big, -big, big, -big))
    for i in range(6):
        mmst[pl.ds(i * _L, _L)] = lax.bitcast_convert_type(mm[i], jnp.int32)
    pltpu.sync_copy(mmst, mmsh.at[pl.ds((lb * _T + q) * 96, 96)])
    plsc.subcore_barrier()
    pltpu.sync_copy(mmsh.at[pl.ds(lb * _T * 96, _T * 96)],
                    hist4.at[pl.ds(0, _T * 96)])
    gl = []
    for i in range(6):
        red = jnp.minimum if (i % 2 == 0) else jnp.maximum
        acc = _f32(hist4[pl.ds(i * _L, _L)])
        for t in range(1, _T):
            acc = red(acc, _f32(hist4[pl.ds(t * 96 + i * _L, _L)]))
        gl.append(jnp.min(acc) if (i % 2 == 0) else jnp.max(acc))
    bmin_x, bmax_x, bmin_y, bmax_y, bmin_z, bmax_z = gl

    mnxv = jnp.full((_L,), bmin_x, jnp.float32)
    rxv = jnp.full((_L,), bmax_x - bmin_x, jnp.float32)
    mnyv = jnp.full((_L,), bmin_y, jnp.float32)
    ryv = jnp.full((_L,), bmax_y - bmin_y, jnp.float32)
    mnzv = jnp.full((_L,), bmin_z, jnp.float32)
    rzv = jnp.full((_L,), bmax_z - bmin_z, jnp.float32)

    def zero_hist(i, _):
        hist[pl.ds(i * _L, _L)] = zeros
        return 0

    lax.fori_loop(0, _RADIX // _L, zero_hist, 0)

    def quant(j, _):
        s = j * _L
        fx = _f32(qx[pl.ds(s, _L)])
        fy = _f32(qy[pl.ds(s, _L)])
        fz = _f32(qz[pl.ds(s, _L)])
        px = (_HMAX * (fx - mnxv) / rxv).astype(jnp.int32)
        py = (_HMAX * (fy - mnyv) / ryv).astype(jnp.int32)
        pz = (_HMAX * (fz - mnzv) / rzv).astype(jnp.int32)
        code = (_spread_bits(px) | (_spread_bits(py) << 1)
                | (_spread_bits(pz) << 2))
        qx[pl.ds(s, _L)] = code
        plsc.addupdate_scatter(hist, [code & (_RADIX - 1)], ones)
        return 0

    lax.fori_loop(0, _QCHUNK, quant, 0)

    pltpu.sync_copy(qx, zsh.at[pl.ds(lb * _N + qbase, _Q)])
    plsc.subcore_barrier()
    pltpu.sync_copy(zsh.at[pl.ds(lb * _N, _N)], zbuf)

    qsel = jnp.full((_L,), q, jnp.int32)

    def prefix_from_hist4():
        def prefix(t, carry):
            d0 = t * _L
            h0 = hist4[pl.ds(d0, _L)]
            h1 = hist4[pl.ds(_RADIX + d0, _L)]
            h2 = hist4[pl.ds(2 * _RADIX + d0, _L)]
            h3 = hist4[pl.ds(3 * _RADIX + d0, _L)]
            tot = h0 + h1 + h2 + h3
            part = (jnp.where(qsel > 0, h0, zeros)
                    + jnp.where(qsel > 1, h1, zeros)
                    + jnp.where(qsel > 2, h2, zeros))
            incl = plsc.cumsum(tot)
            offs[pl.ds(d0, _L)] = (incl - tot + part
                                   + jnp.full((_L,), carry, jnp.int32))
            hist[pl.ds(d0, _L)] = zeros
            return carry + jnp.max(incl)

        lax.fori_loop(0, _RADIX // _L, prefix, jnp.int32(0))

    def exchange_hist():
        pltpu.sync_copy(hist, hsh.at[pl.ds((lb * _T + q) * _RADIX, _RADIX)])
        plsc.subcore_barrier()
        pltpu.sync_copy(hsh.at[pl.ds(lb * _T * _RADIX, _T * _RADIX)], hist4)

    def hist_sweep(shift):
        def histo(j, _):
            s = j * _L
            old = perm[pl.ds(s, _L)]
            keys = plsc.load_gather(zbuf, [old])
            qz[pl.ds(s, _L)] = keys
            d = (keys >> shift) & (_RADIX - 1)
            plsc.addupdate_scatter(hist, [d], ones)
            return 0

        lax.fori_loop(0, _QCHUNK, histo, 0)

    def perm_sweep(shift, first, dst_sh):
        nbat = _Q // _GB
        nrow = _GB // 128

        def batch(m, _):
            def group(g, _):
                j = m * (_GB // _L) + g
                s = j * _L
                if first:
                    old = qbase + s + iota
                    keys = qx[pl.ds(s, _L)]
                else:
                    old = perm[pl.ds(s, _L)]
                    keys = qz[pl.ds(s, _L)]
                d = (keys >> shift) & (_RADIX - 1)
                base = plsc.load_gather(offs, [d])
                cnt, last = plsc.scan_count(d)
                dest = base + cnt - 1
                row = g >> 3
                col = (g & 7) * _L
                svals[row, pl.ds(col, _L)] = old
                sidx[row, pl.ds(col, _L)] = dest + lb * _N
                plsc.store_scatter(offs, [d], dest + 1, mask=last)
                return 0

            lax.fori_loop(0, _GB // _L, group, 0)
            copies = [pltpu.async_copy(svals.at[r], dst_sh.at[sidx.at[r]],
                                       sem) for r in range(nrow)]
            for cp in copies:
                cp.wait()
            return 0

        lax.fori_loop(0, nbat, batch, 0)
        plsc.subcore_barrier()

    exchange_hist()
    prefix_from_hist4()
    perm_sweep(0, True, psha)
    pltpu.sync_copy(psha.at[pl.ds(lb * _N + qbase, _Q)], perm)
    hist_sweep(10)
    exchange_hist()
    prefix_from_hist4()
    perm_sweep(10, False, pshb)
    pltpu.sync_copy(pshb.at[pl.ds(lb * _N + qbase, _Q)], perm)
    hist_sweep(20)
    exchange_hist()
    prefix_from_hist4()
    perm_sweep(20, False, psha)
    pltpu.sync_copy(psha.at[pl.ds(lb * _N + qbase, _Q)], perm)

    for c, dst in zip(range(3), (qx, qy, qz)):
        pltpu.sync_copy(xt.at[b, c], zbuf)

        def apply_perm(j, _):
            s = j * _L
            old = perm[pl.ds(s, _L)]
            dst[pl.ds(s, _L)] = plsc.load_gather(zbuf, [old])
            return 0

        lax.fori_loop(0, _QCHUNK, apply_perm, 0)

    i3 = iota * 3

    def interleave(j, _):
        s = j * _L
        s3 = j * (_L * 3)
        plsc.store_scatter(qint, [i3 + s3], qx[pl.ds(s, _L)])
        plsc.store_scatter(qint, [i3 + (s3 + 1)], qy[pl.ds(s, _L)])
        plsc.store_scatter(qint, [i3 + (s3 + 2)], qz[pl.ds(s, _L)])
        return 0

    lax.fori_loop(0, _QCHUNK, interleave, 0)
    pltpu.sync_copy(qint, ot.at[b, pl.ds(qbase * 3, _Q * 3)])


@jax.jit
def kernel(x):
    xt = lax.bitcast_convert_type(jnp.swapaxes(x, 1, 2), jnp.int32)
    mesh = plsc.VectorSubcoreMesh(core_axis_name="c", subcore_axis_name="s")
    run = pl.kernel(
        _sc_body,
        out_type=jax.ShapeDtypeStruct((_NB, _N * 3), jnp.int32),
        mesh=mesh,
        compiler_params=pltpu.CompilerParams(needs_layout_passes=False,
                                             use_tc_tiling_on_sc=False),
        scratch_types=[
            pltpu.VMEM_SHARED((4 * _N,), jnp.int32),
            pltpu.VMEM_SHARED((4 * _N,), jnp.int32),
            pltpu.VMEM_SHARED((4 * _N,), jnp.int32),
            pltpu.VMEM_SHARED((4 * _T * _RADIX,), jnp.int32),
            pltpu.VMEM_SHARED((4 * _T * 96,), jnp.int32),
            pltpu.VMEM((_N,), jnp.int32),
            pltpu.VMEM((_Q,), jnp.int32),
            pltpu.VMEM((_Q,), jnp.int32),
            pltpu.VMEM((_Q,), jnp.int32),
            pltpu.VMEM((_Q * 3,), jnp.int32),
            pltpu.VMEM((_Q,), jnp.int32),
            pltpu.VMEM((_RADIX,), jnp.int32),
            pltpu.VMEM((_RADIX,), jnp.int32),
            pltpu.VMEM((_T * _RADIX,), jnp.int32),
            pltpu.VMEM((96,), jnp.int32),
            pltpu.VMEM((_GB // 128, 128), jnp.int32),
            pltpu.VMEM((_GB // 128, 128), jnp.int32),
            pltpu.SemaphoreType.DMA,
        ],
    )
    outi = run(xt)
    return lax.bitcast_convert_type(jnp.reshape(outi, (_NB, _N, 3)),
                                    jnp.float32)


# speedup vs baseline: 3.5254x; 2.2536x over previous
import jax
import jax.numpy as jnp
from jax import lax
from jax.experimental import pallas as pl
from jax.experimental.pallas import tpu as pltpu
from jax.experimental.pallas import tpu_sc as plsc

_NB = 8
_N = 32768
_L = 16
_T = 4
_Q = _N // _T
_QCHUNK = _Q // _L
_NCHUNK = _N // _L
_RADIX = 1024
_GB = 1024
_HMAX = float((1 << 10) - 1)


def _spread_bits(n):
    n = n & 0x000003FF
    n = (n | (n << 16)) & 0x030000FF
    n = (n | (n << 8)) & 0x0300F00F
    n = (n | (n << 4)) & 0x030C30C3
    n = (n | (n << 2)) & 0x09249249
    return n


def _f32(v):
    return lax.bitcast_convert_type(v, jnp.float32)


def _sc_body(xt, ot, zsh, psha, pshb, hsh, mmsh,
             zbuf, qx, qy, qz, perm, hist, offs, hist4, mmst, svals, sidx,
             sem):
    w = lax.axis_index("c") * 16 + lax.axis_index("s")
    b = w >> 2
    lb = b & 3
    q = w & 3
    iota = lax.iota(jnp.int32, _L)
    zeros = jnp.zeros((_L,), jnp.int32)
    ones = jnp.ones((_L,), jnp.int32)
    qbase = q * _Q

    pltpu.sync_copy(xt.at[b, 0, pl.ds(qbase, _Q)], qx)
    pltpu.sync_copy(xt.at[b, 1, pl.ds(qbase, _Q)], qy)
    pltpu.sync_copy(xt.at[b, 2, pl.ds(qbase, _Q)], qz)

    def minmax(j, mm):
        mnx, mxx, mny, mxy, mnz, mxz = mm
        s = j * _L
        fx = _f32(qx[pl.ds(s, _L)])
        fy = _f32(qy[pl.ds(s, _L)])
        fz = _f32(qz[pl.ds(s, _L)])
        return (jnp.minimum(mnx, fx), jnp.maximum(mxx, fx),
                jnp.minimum(mny, fy), jnp.maximum(mxy, fy),
                jnp.minimum(mnz, fz), jnp.maximum(mxz, fz))

    big = jnp.full((_L,), jnp.inf, jnp.float32)
    mm = lax.fori_loop(0, _QCHUNK, minmax, (big, -big, big, -big, big, -big))
    for i in range(6):
        mmst[pl.ds(i * _L, _L)] = lax.bitcast_convert_type(mm[i], jnp.int32)
    pltpu.sync_copy(mmst, mmsh.at[pl.ds((lb * _T + q) * 96, 96)])
    plsc.subcore_barrier()
    pltpu.sync_copy(mmsh.at[pl.ds(lb * _T * 96, _T * 96)],
                    hist4.at[pl.ds(0, _T * 96)])
    gl = []
    for i in range(6):
        red = jnp.minimum if (i % 2 == 0) else jnp.maximum
        acc = _f32(hist4[pl.ds(i * _L, _L)])
        for t in range(1, _T):
            acc = red(acc, _f32(hist4[pl.ds(t * 96 + i * _L, _L)]))
        gl.append(jnp.min(acc) if (i % 2 == 0) else jnp.max(acc))
    bmin_x, bmax_x, bmin_y, bmax_y, bmin_z, bmax_z = gl

    mnxv = jnp.full((_L,), bmin_x, jnp.float32)
    rxv = jnp.full((_L,), bmax_x - bmin_x, jnp.float32)
    mnyv = jnp.full((_L,), bmin_y, jnp.float32)
    ryv = jnp.full((_L,), bmax_y - bmin_y, jnp.float32)
    mnzv = jnp.full((_L,), bmin_z, jnp.float32)
    rzv = jnp.full((_L,), bmax_z - bmin_z, jnp.float32)

    def zero_hist(i, _):
        hist[pl.ds(i * _L, _L)] = zeros
        return 0

    lax.fori_loop(0, _RADIX // _L, zero_hist, 0)

    def quant(j, _):
        s = j * _L
        fx = _f32(qx[pl.ds(s, _L)])
        fy = _f32(qy[pl.ds(s, _L)])
        fz = _f32(qz[pl.ds(s, _L)])
        px = (_HMAX * (fx - mnxv) / rxv).astype(jnp.int32)
        py = (_HMAX * (fy - mnyv) / ryv).astype(jnp.int32)
        pz = (_HMAX * (fz - mnzv) / rzv).astype(jnp.int32)
        code = (_spread_bits(px) | (_spread_bits(py) << 1)
                | (_spread_bits(pz) << 2))
        qx[pl.ds(s, _L)] = code
        plsc.addupdate_scatter(hist, [code & (_RADIX - 1)], ones)
        return 0

    lax.fori_loop(0, _QCHUNK, quant, 0)

    pltpu.sync_copy(qx, zsh.at[pl.ds(lb * _N + qbase, _Q)])
    plsc.subcore_barrier()
    pltpu.sync_copy(zsh.at[pl.ds(lb * _N, _N)], zbuf)

    qsel = jnp.full((_L,), q, jnp.int32)

    def prefix_from_hist4():
        def prefix(t, carry):
            d0 = t * _L
            h0 = hist4[pl.ds(d0, _L)]
            h1 = hist4[pl.ds(_RADIX + d0, _L)]
            h2 = hist4[pl.ds(2 * _RADIX + d0, _L)]
            h3 = hist4[pl.ds(3 * _RADIX + d0, _L)]
            tot = h0 + h1 + h2 + h3
            part = (jnp.where(qsel > 0, h0, zeros)
                    + jnp.where(qsel > 1, h1, zeros)
                    + jnp.where(qsel > 2, h2, zeros))
            incl = plsc.cumsum(tot)
            offs[pl.ds(d0, _L)] = (incl - tot + part
                                   + jnp.full((_L,), carry, jnp.int32))
            hist[pl.ds(d0, _L)] = zeros
            return carry + jnp.max(incl)

        lax.fori_loop(0, _RADIX // _L, prefix, jnp.int32(0))

    def exchange_hist():
        pltpu.sync_copy(hist, hsh.at[pl.ds((lb * _T + q) * _RADIX, _RADIX)])
        plsc.subcore_barrier()
        pltpu.sync_copy(hsh.at[pl.ds(lb * _T * _RADIX, _T * _RADIX)], hist4)

    def hist_sweep(shift):
        def histo(j, _):
            s = j * _L
            old = perm[pl.ds(s, _L)]
            keys = plsc.load_gather(zbuf, [old])
            qz[pl.ds(s, _L)] = keys
            d = (keys >> shift) & (_RADIX - 1)
            plsc.addupdate_scatter(hist, [d], ones)
            return 0

        lax.fori_loop(0, _QCHUNK, histo, 0)

    def perm_sweep(shift, first, dst_sh):
        nbat = _Q // _GB
        nrow = _GB // 128

        def batch(m, _):
            def group(g, _):
                j = m * (_GB // _L) + g
                s = j * _L
                if first:
                    old = qbase + s + iota
                    keys = qx[pl.ds(s, _L)]
                else:
                    old = perm[pl.ds(s, _L)]
                    keys = qz[pl.ds(s, _L)]
                d = (keys >> shift) & (_RADIX - 1)
                base = plsc.load_gather(offs, [d])
                cnt, last = plsc.scan_count(d)
                dest = base + cnt - 1
                row = g >> 3
                col = (g & 7) * _L
                svals[row, pl.ds(col, _L)] = old
                sidx[row, pl.ds(col, _L)] = dest + lb * _N
                plsc.store_scatter(offs, [d], dest + 1, mask=last)
                return 0

            lax.fori_loop(0, _GB // _L, group, 0)
            copies = [pltpu.async_copy(svals.at[r], dst_sh.at[sidx.at[r]],
                                       sem) for r in range(nrow)]
            for cp in copies:
                cp.wait()
            return 0

        lax.fori_loop(0, nbat, batch, 0)
        plsc.subcore_barrier()

    exchange_hist()
    prefix_from_hist4()
    perm_sweep(0, True, psha)
    pltpu.sync_copy(psha.at[pl.ds(lb * _N + qbase, _Q)], perm)
    hist_sweep(10)
    exchange_hist()
    prefix_from_hist4()
    perm_sweep(10, False, pshb)
    pltpu.sync_copy(pshb.at[pl.ds(lb * _N + qbase, _Q)], perm)
    hist_sweep(20)
    exchange_hist()
    prefix_from_hist4()
    perm_sweep(20, False, psha)
    pltpu.sync_copy(psha.at[pl.ds(lb * _N + qbase, _Q)], perm)

    for c in range(3):
        pltpu.sync_copy(xt.at[b, c], zbuf)

        def apply_perm(j, _):
            s = j * _L
            old = perm[pl.ds(s, _L)]
            qy[pl.ds(s, _L)] = plsc.load_gather(zbuf, [old])
            return 0

        lax.fori_loop(0, _QCHUNK, apply_perm, 0)
        pltpu.sync_copy(qy, ot.at[b, c, pl.ds(qbase, _Q)])


@jax.jit
def kernel(x):
    xt = lax.bitcast_convert_type(jnp.swapaxes(x, 1, 2), jnp.int32)
    mesh = plsc.VectorSubcoreMesh(core_axis_name="c", subcore_axis_name="s")
    run = pl.kernel(
        _sc_body,
        out_type=jax.ShapeDtypeStruct((_NB, 3, _N), jnp.int32),
        mesh=mesh,
        compiler_params=pltpu.CompilerParams(needs_layout_passes=False,
                                             use_tc_tiling_on_sc=False),
        scratch_types=[
            pltpu.VMEM_SHARED((4 * _N,), jnp.int32),
            pltpu.VMEM_SHARED((4 * _N,), jnp.int32),
            pltpu.VMEM_SHARED((4 * _N,), jnp.int32),
            pltpu.VMEM_SHARED((4 * _T * _RADIX,), jnp.int32),
            pltpu.VMEM_SHARED((4 * _T * 96,), jnp.int32),
            pltpu.VMEM((_N,), jnp.int32),
            pltpu.VMEM((_Q,), jnp.int32),
            pltpu.VMEM((_Q,), jnp.int32),
            pltpu.VMEM((_Q,), jnp.int32),
            pltpu.VMEM((_Q,), jnp.int32),
            pltpu.VMEM((_RADIX,), jnp.int32),
            pltpu.VMEM((_RADIX,), jnp.int32),
            pltpu.VMEM((_T * _RADIX,), jnp.int32),
            pltpu.VMEM((96,), jnp.int32),
            pltpu.VMEM((_GB // 128, 128), jnp.int32),
            pltpu.VMEM((_GB // 128, 128), jnp.int32),
            pltpu.SemaphoreType.DMA,
        ],
    )
    outt = run(xt)
    return jnp.swapaxes(lax.bitcast_convert_type(outt, jnp.float32), 1, 2)
